# Initial kernel scaffold; baseline (speedup 1.0000x reference)
import jax
import jax.numpy as jnp
from jax.experimental import pallas as pl


def kernel(userIdx, servIdx, eu_gmf, eu_mlp, ei_gmf, ei_mlp, W1, b1, W2, b2, W3, b3, Wp, bp):
    raise NotImplementedError("write your pallas kernel here")


# speedup vs baseline: 2.5750x; 2.5750x over previous
import functools

import jax
import jax.numpy as jnp
from jax import lax
from jax.experimental import pallas as pl
from jax.experimental.pallas import tpu as pltpu
from jax.experimental.pallas import tpu_sc as plsc

NC = 2
NS = 16
NW = NC * NS
CH = 128

BM = 1024


def _sc_gather(uidx, sidx, utab, itab):
    B = uidx.shape[0]
    W = utab.shape[1]
    b_per_w = B // NW
    n_ch = b_per_w // CH
    mesh = plsc.VectorSubcoreMesh(core_axis_name="c", subcore_axis_name="s")

    @functools.partial(
        pl.kernel,
        mesh=mesh,
        out_type=(
            jax.ShapeDtypeStruct((B, W), jnp.float32),
            jax.ShapeDtypeStruct((B, W), jnp.float32),
        ),
        scratch_types=(
            pltpu.VMEM((CH,), jnp.int32),
            pltpu.VMEM((CH,), jnp.int32),
            pltpu.VMEM((CH, W), jnp.float32),
            pltpu.VMEM((CH, W), jnp.float32),
            pltpu.SemaphoreType.DMA,
        ),
    )
    def gather_k(uidx_h, sidx_h, ut_h, it_h,
                 u_o, i_o,
                 uidx_v, sidx_v, u_v, i_v, sem):
        wid = lax.axis_index("s") * NC + lax.axis_index("c")
        base = wid * b_per_w
        for c in range(n_ch):
            off = base + c * CH
            pltpu.sync_copy(uidx_h.at[pl.ds(off, CH)], uidx_v)
            pltpu.sync_copy(sidx_h.at[pl.ds(off, CH)], sidx_v)
            cp1 = pltpu.async_copy(ut_h.at[uidx_v], u_v, sem)
            cp2 = pltpu.async_copy(it_h.at[sidx_v], i_v, sem)
            cp1.wait()
            cp2.wait()
            pltpu.sync_copy(u_v, u_o.at[pl.ds(off, CH)])
            pltpu.sync_copy(i_v, i_o.at[pl.ds(off, CH)])

    return gather_k(uidx, sidx, utab, itab)


def _mlp_body(uref, iref, w1a, w1b, b1, w2, b2, w3, b3, wpg, wph, bp, out):
    u = uref[...]
    i = iref[...]
    um, ug = u[:, :256], u[:, 256:320]
    im, ig = i[:, :256], i[:, 256:320]
    h = jnp.dot(um, w1a[...], preferred_element_type=jnp.float32)
    h = h + jnp.dot(im, w1b[...], preferred_element_type=jnp.float32)
    h = jnp.maximum(h + b1[...], 0.0)
    h = jnp.maximum(jnp.dot(h, w2[...], preferred_element_type=jnp.float32) + b2[...], 0.0)
    h = jnp.maximum(jnp.dot(h, w3[...], preferred_element_type=jnp.float32) + b3[...], 0.0)
    g = ug * ig
    p = jnp.sum(g * wpg[...], axis=1) + jnp.sum(h * wph[...], axis=1) + bp[0, 0]
    out[0, 0, :] = p


def kernel(userIdx, servIdx, eu_gmf, eu_mlp, ei_gmf, ei_mlp,
           W1, b1, W2, b2, W3, b3, Wp, bp):
    B = userIdx.shape[0]
    uidx = userIdx.astype(jnp.int32)
    sidx = servIdx.astype(jnp.int32)

    Dm = eu_mlp.shape[1]
    Dg = eu_gmf.shape[1]
    DgP = 128
    Wt = Dm + DgP

    utab = jnp.concatenate(
        [eu_mlp,
         jnp.pad(eu_gmf, ((0, eu_mlp.shape[0] - eu_gmf.shape[0]), (0, DgP - Dg)))],
        axis=1)
    itab = jnp.concatenate(
        [ei_mlp,
         jnp.pad(ei_gmf, ((0, ei_mlp.shape[0] - ei_gmf.shape[0]), (0, DgP - Dg)))],
        axis=1)

    urows, irows = _sc_gather(uidx, sidx, utab, itab)

    H1 = W1.shape[1]
    H2 = W2.shape[1]
    H3 = W3.shape[1]
    nblk = B // BM

    full = lambda i: (0, 0)
    out = pl.pallas_call(
        _mlp_body,
        grid=(nblk,),
        in_specs=[
            pl.BlockSpec((BM, Wt), lambda i: (i, 0)),
            pl.BlockSpec((BM, Wt), lambda i: (i, 0)),
            pl.BlockSpec((Dm, H1), full),
            pl.BlockSpec((Dm, H1), full),
            pl.BlockSpec((1, H1), full),
            pl.BlockSpec((H1, H2), full),
            pl.BlockSpec((1, H2), full),
            pl.BlockSpec((H2, H3), full),
            pl.BlockSpec((1, H3), full),
            pl.BlockSpec((1, Dg), full),
            pl.BlockSpec((1, H3), full),
            pl.BlockSpec((1, 1), full),
        ],
        out_specs=pl.BlockSpec((1, 1, BM), lambda i: (i, 0, 0)),
        out_shape=jax.ShapeDtypeStruct((nblk, 1, BM), jnp.float32),
    )(urows, irows,
      W1[:Dm], W1[Dm:], b1.reshape(1, H1),
      W2, b2.reshape(1, H2),
      W3, b3.reshape(1, H3),
      Wp[:Dg].reshape(1, Dg), Wp[Dg:].reshape(1, H3),
      bp.reshape(1, 1))
    return out.reshape(-1)


# speedup vs baseline: 2.7384x; 1.0634x over previous
import functools

import jax
import jax.numpy as jnp
from jax import lax
from jax.experimental import pallas as pl
from jax.experimental.pallas import tpu as pltpu
from jax.experimental.pallas import tpu_sc as plsc

NC = 2
NS = 16
NW = NC * NS
CH = 64

BM = 1024
L = 16


def _precompute(eu_mlp, ei_mlp, w1a, w1b):
    def body(eum, eim, wa, wb, pu_o, pi_o):
        pu_o[...] = jnp.dot(eum[...], wa[...], preferred_element_type=jnp.float32)
        pi_o[...] = jnp.dot(eim[...], wb[...], preferred_element_type=jnp.float32)

    return pl.pallas_call(
        body,
        out_shape=(
            jax.ShapeDtypeStruct(eu_mlp.shape, jnp.float32),
            jax.ShapeDtypeStruct(ei_mlp.shape, jnp.float32),
        ),
    )(eu_mlp, ei_mlp, w1a, w1b)


def _sc_fuse(uidx, sidx, utab, itab):
    B = uidx.shape[0]
    W = utab.shape[1]
    b_per_w = B // NW
    n_ch = b_per_w // CH
    mesh = plsc.VectorSubcoreMesh(core_axis_name="c", subcore_axis_name="s")

    @functools.partial(
        pl.kernel,
        mesh=mesh,
        out_type=jax.ShapeDtypeStruct((B, W), jnp.float32),
        scratch_types=(
            pltpu.VMEM((b_per_w,), jnp.int32),
            pltpu.VMEM((b_per_w,), jnp.int32),
            pltpu.VMEM((CH, W), jnp.float32),
            pltpu.VMEM((CH, W), jnp.float32),
            pltpu.VMEM((CH, W), jnp.float32),
            pltpu.VMEM((CH, W), jnp.float32),
            pltpu.SemaphoreType.DMA,
            pltpu.SemaphoreType.DMA,
            pltpu.SemaphoreType.DMA,
            pltpu.SemaphoreType.DMA,
        ),
    )
    def fuse_k(uidx_h, sidx_h, ut_h, it_h, o_h,
               uidx_v, sidx_v, u0, i0, u1, i1, g0, g1, w0, w1):
        wid = lax.axis_index("s") * NC + lax.axis_index("c")
        base = wid * b_per_w
        cpu = pltpu.async_copy(uidx_h.at[pl.ds(base, b_per_w)], uidx_v, w0)
        cpi = pltpu.async_copy(sidx_h.at[pl.ds(base, b_per_w)], sidx_v, w0)
        cpu.wait()
        cpi.wait()

        ubuf = (u0, u1)
        ibuf = (i0, i1)
        gsem = (g0, g1)
        wsem = (w0, w1)

        def fire(c):
            k = c % 2
            gu = pltpu.async_copy(
                ut_h.at[uidx_v.at[pl.ds(c * CH, CH)]], ubuf[k], gsem[k])
            gi = pltpu.async_copy(
                it_h.at[sidx_v.at[pl.ds(c * CH, CH)]], ibuf[k], gsem[k])
            return gu, gi

        def compute(c):
            k = c % 2
            u, i = ubuf[k], ibuf[k]

            def row(r, _):
                for j in range(16):
                    sl = pl.ds(j * L, L)
                    u[r, sl] = u[r, sl] + i[r, sl]
                for j in range(4):
                    sl = pl.ds(256 + j * L, L)
                    u[r, sl] = u[r, sl] * i[r, sl]
                return 0

            lax.fori_loop(0, CH, row, 0)

        wb = [None, None]
        cur = fire(0)
        for c in range(n_ch):
            k = c % 2
            nxt = None
            if c + 1 < n_ch:
                if wb[(c + 1) % 2] is not None:
                    wb[(c + 1) % 2].wait()
                nxt = fire(c + 1)
            cur[0].wait()
            cur[1].wait()
            compute(c)
            wb[k] = pltpu.async_copy(
                ubuf[k], o_h.at[pl.ds(base + c * CH, CH)], wsem[k])
            cur = nxt
        wb[0].wait()
        wb[1].wait()

    return fuse_k(uidx, sidx, utab, itab)


def _finish_body(hg_ref, b1, w2, b2, w3, b3, wpg, wph, bp, out):
    hg = hg_ref[...]
    h = jnp.maximum(hg[:, :256] + b1[...], 0.0)
    h = jnp.maximum(jnp.dot(h, w2[...], preferred_element_type=jnp.float32) + b2[...], 0.0)
    h = jnp.maximum(jnp.dot(h, w3[...], preferred_element_type=jnp.float32) + b3[...], 0.0)
    p = jnp.sum(hg[:, 256:320] * wpg[...], axis=1) + jnp.sum(h * wph[...], axis=1) + bp[0, 0]
    out[0, 0, :] = p


def kernel(userIdx, servIdx, eu_gmf, eu_mlp, ei_gmf, ei_mlp,
           W1, b1, W2, b2, W3, b3, Wp, bp):
    B = userIdx.shape[0]
    uidx = userIdx.astype(jnp.int32)
    sidx = servIdx.astype(jnp.int32)

    Dm = eu_mlp.shape[1]
    Dg = eu_gmf.shape[1]
    DgP = 128
    Wt = Dm + DgP

    PU, PI = _precompute(eu_mlp, ei_mlp, W1[:Dm], W1[Dm:])

    utab = jnp.concatenate(
        [PU,
         jnp.pad(eu_gmf, ((0, eu_mlp.shape[0] - eu_gmf.shape[0]), (0, DgP - Dg)))],
        axis=1)
    itab = jnp.concatenate(
        [PI,
         jnp.pad(ei_gmf, ((0, ei_mlp.shape[0] - ei_gmf.shape[0]), (0, DgP - Dg)))],
        axis=1)

    hg = _sc_fuse(uidx, sidx, utab, itab)

    H1 = W1.shape[1]
    H2 = W2.shape[1]
    H3 = W3.shape[1]
    nblk = B // BM

    full = lambda i: (0, 0)
    out = pl.pallas_call(
        _finish_body,
        grid=(nblk,),
        in_specs=[
            pl.BlockSpec((BM, Wt), lambda i: (i, 0)),
            pl.BlockSpec((1, H1), full),
            pl.BlockSpec((H1, H2), full),
            pl.BlockSpec((1, H2), full),
            pl.BlockSpec((H2, H3), full),
            pl.BlockSpec((1, H3), full),
            pl.BlockSpec((1, Dg), full),
            pl.BlockSpec((1, H3), full),
            pl.BlockSpec((1, 1), full),
        ],
        out_specs=pl.BlockSpec((1, 1, BM), lambda i: (i, 0, 0)),
        out_shape=jax.ShapeDtypeStruct((nblk, 1, BM), jnp.float32),
    )(hg,
      b1.reshape(1, H1),
      W2, b2.reshape(1, H2),
      W3, b3.reshape(1, H3),
      Wp[:Dg].reshape(1, Dg), Wp[Dg:].reshape(1, H3),
      bp.reshape(1, 1))
    return out.reshape(-1)


# speedup vs baseline: 2.8486x; 1.0402x over previous
import functools

import jax
import jax.numpy as jnp
from jax import lax
from jax.experimental import pallas as pl
from jax.experimental.pallas import tpu as pltpu
from jax.experimental.pallas import tpu_sc as plsc

NC = 2
NS = 16
NW = NC * NS
CH = 64

BM = 1024
L = 16
NSPLIT = 2


def _precompute(eu_mlp, ei_mlp, eug_p, ei_gmf, w1a, w1b):
    Wt = 384

    def body(eum, eim, eug, eig, wa, wb, ut_o, it_o):
        ut_o[:, :256] = jnp.dot(eum[...], wa[...], preferred_element_type=jnp.float32)
        ut_o[:, 256:320] = eug[...]
        ut_o[:, 320:] = jnp.zeros((eum.shape[0], 64), jnp.float32)
        it_o[:, :256] = jnp.dot(eim[...], wb[...], preferred_element_type=jnp.float32)
        it_o[:, 256:320] = eig[...]
        it_o[:, 320:] = jnp.zeros((eim.shape[0], 64), jnp.float32)

    return pl.pallas_call(
        body,
        out_shape=(
            jax.ShapeDtypeStruct((eu_mlp.shape[0], Wt), jnp.float32),
            jax.ShapeDtypeStruct((ei_mlp.shape[0], Wt), jnp.float32),
        ),
    )(eu_mlp, ei_mlp, eug_p, ei_gmf, w1a, w1b)


def _sc_fuse(uidx, sidx, utab, itab):
    B = uidx.shape[0]
    W = utab.shape[1]
    b_per_w = B // NW
    n_ch = b_per_w // CH
    mesh = plsc.VectorSubcoreMesh(core_axis_name="c", subcore_axis_name="s")

    @functools.partial(
        pl.kernel,
        mesh=mesh,
        out_type=jax.ShapeDtypeStruct((B, W), jnp.float32),
        scratch_types=(
            pltpu.VMEM((b_per_w,), jnp.int32),
            pltpu.VMEM((b_per_w,), jnp.int32),
            pltpu.VMEM((CH, W), jnp.float32),
            pltpu.VMEM((CH, W), jnp.float32),
            pltpu.VMEM((CH, W), jnp.float32),
            pltpu.VMEM((CH, W), jnp.float32),
            pltpu.SemaphoreType.DMA,
            pltpu.SemaphoreType.DMA,
            pltpu.SemaphoreType.DMA,
            pltpu.SemaphoreType.DMA,
        ),
    )
    def fuse_k(uidx_h, sidx_h, ut_h, it_h, o_h,
               uidx_v, sidx_v, u0, i0, u1, i1, g0, g1, w0, w1):
        wid = lax.axis_index("s") * NC + lax.axis_index("c")
        base = wid * b_per_w
        cpu = pltpu.async_copy(uidx_h.at[pl.ds(base, b_per_w)], uidx_v, w0)
        cpi = pltpu.async_copy(sidx_h.at[pl.ds(base, b_per_w)], sidx_v, w0)
        cpu.wait()
        cpi.wait()

        ubuf = (u0, u1)
        ibuf = (i0, i1)
        gsem = (g0, g1)
        wsem = (w0, w1)

        def fire(c):
            k = c % 2
            gu = pltpu.async_copy(
                ut_h.at[uidx_v.at[pl.ds(c * CH, CH)]], ubuf[k], gsem[k])
            gi = pltpu.async_copy(
                it_h.at[sidx_v.at[pl.ds(c * CH, CH)]], ibuf[k], gsem[k])
            return gu, gi

        def compute(c):
            k = c % 2
            u, i = ubuf[k], ibuf[k]

            def row(r, _):
                for j in range(16):
                    sl = pl.ds(j * L, L)
                    u[r, sl] = u[r, sl] + i[r, sl]
                for j in range(4):
                    sl = pl.ds(256 + j * L, L)
                    u[r, sl] = u[r, sl] * i[r, sl]
                return 0

            lax.fori_loop(0, CH, row, 0)

        wb = [None, None]
        cur = fire(0)
        for c in range(n_ch):
            k = c % 2
            nxt = None
            if c + 1 < n_ch:
                if wb[(c + 1) % 2] is not None:
                    wb[(c + 1) % 2].wait()
                nxt = fire(c + 1)
            cur[0].wait()
            cur[1].wait()
            compute(c)
            wb[k] = pltpu.async_copy(
                ubuf[k], o_h.at[pl.ds(base + c * CH, CH)], wsem[k])
            cur = nxt
        wb[0].wait()
        if wb[1] is not None:
            wb[1].wait()

    return fuse_k(uidx, sidx, utab, itab)


def _finish_body(hg_ref, b1, w2, b2, w3, b3, wpg, wph, bp, out):
    hg = hg_ref[...]
    h = jnp.maximum(hg[:, :256] + b1[...], 0.0)
    h = jnp.maximum(jnp.dot(h, w2[...], preferred_element_type=jnp.float32) + b2[...], 0.0)
    h = jnp.maximum(jnp.dot(h, w3[...], preferred_element_type=jnp.float32) + b3[...], 0.0)
    p = jnp.sum(hg[:, 256:320] * wpg[...], axis=1) + jnp.sum(h * wph[...], axis=1) + bp[0, 0]
    out[0, 0, :] = p


def _finish(hg, b1, W2, b2, W3, b3, wpg, wph, bp):
    B = hg.shape[0]
    Wt = hg.shape[1]
    H1, H2, H3 = 256, 128, 64
    nblk = B // BM
    full = lambda i: (0, 0)
    out = pl.pallas_call(
        _finish_body,
        grid=(nblk,),
        in_specs=[
            pl.BlockSpec((BM, Wt), lambda i: (i, 0)),
            pl.BlockSpec((1, H1), full),
            pl.BlockSpec((H1, H2), full),
            pl.BlockSpec((1, H2), full),
            pl.BlockSpec((H2, H3), full),
            pl.BlockSpec((1, H3), full),
            pl.BlockSpec((1, 64), full),
            pl.BlockSpec((1, H3), full),
            pl.BlockSpec((1, 1), full),
        ],
        out_specs=pl.BlockSpec((1, 1, BM), lambda i: (i, 0, 0)),
        out_shape=jax.ShapeDtypeStruct((nblk, 1, BM), jnp.float32),
    )(hg, b1, W2, b2, W3, b3, wpg, wph, bp)
    return out.reshape(-1)


def kernel(userIdx, servIdx, eu_gmf, eu_mlp, ei_gmf, ei_mlp,
           W1, b1, W2, b2, W3, b3, Wp, bp):
    B = userIdx.shape[0]
    uidx = userIdx.astype(jnp.int32)
    sidx = servIdx.astype(jnp.int32)

    Dm = eu_mlp.shape[1]
    Dg = eu_gmf.shape[1]

    eug_p = jnp.pad(eu_gmf, ((0, eu_mlp.shape[0] - eu_gmf.shape[0]), (0, 0)))
    utab, itab = _precompute(eu_mlp, ei_mlp, eug_p, ei_gmf, W1[:Dm], W1[Dm:])

    b1r = b1.reshape(1, -1)
    b2r = b2.reshape(1, -1)
    b3r = b3.reshape(1, -1)
    wpg = Wp[:Dg].reshape(1, Dg)
    wph = Wp[Dg:].reshape(1, -1)
    bpr = bp.reshape(1, 1)

    H = B // NSPLIT
    parts = []
    for s in range(NSPLIT):
        hg = _sc_fuse(uidx[s * H:(s + 1) * H], sidx[s * H:(s + 1) * H], utab, itab)
        parts.append(_finish(hg, b1r, W2, b2r, W3, b3r, wpg, wph, bpr))
    return jnp.concatenate(parts)
